# baseline (speedup 1.0000x reference)
import functools

import numpy as np

import jax
import jax.numpy as jnp
from jax import lax
from jax.experimental import pallas as pl
from jax.experimental.pallas import tpu as pltpu
from jax.experimental.pallas import tpu_sc as plsc

N = 10000
E = 160000
D = 256
T = 2
B = 4
SD = 10

NC = 2
NS = 16
L = 16

NPAD = 10240

NPT = NPAD // NS

CA = 79
EPT_A = CA * 128
EPAD_A = NS * EPT_A

KC = 64
NW = NC * NS
NCH = 79
EPW = NCH * KC
EPAD_C = NW * EPW
NTAB = 10048


def _dde_body(topic_hbm, src_hbm, dst_hbm, xout_hbm,
              src_v, dst_v, xf_v, xr_v, gf_v, gr_v, zero_v, tmp_v,
              accf_sh, accr_sh, semf, semr):
    cid = lax.axis_index("c")
    sid = lax.axis_index("s")
    pltpu.sync_copy(src_hbm.at[sid], src_v)
    pltpu.sync_copy(dst_hbm.at[sid], dst_v)
    for i in range(NPT // L):
        zero_v[pl.ds(i * L, L)] = jnp.zeros((L,), jnp.float32)
    pltpu.sync_copy(topic_hbm.at[cid], xf_v)
    pltpu.sync_copy(topic_hbm.at[cid], xr_v)

    def do_pair(pair, last):
        sl_t = pl.ds(sid * NPT, NPT)
        pltpu.sync_copy(zero_v, accf_sh.at[sl_t])
        pltpu.sync_copy(zero_v, accr_sh.at[sl_t])
        plsc.subcore_barrier()

        def gbody(i, carry):
            for k in range(8):
                sl = pl.ds(k * L, L)
                gf_v[i, sl] = plsc.load_gather(xf_v, [src_v[i, sl]])
                gr_v[i, sl] = plsc.load_gather(xr_v, [dst_v[i, sl]])
            return carry
        lax.fori_loop(0, CA, gbody, 0)

        for j in range(CA):
            pltpu.async_copy(gf_v.at[j], accf_sh.at[dst_v.at[j]], semf,
                             add=True)
            pltpu.async_copy(gr_v.at[j], accr_sh.at[src_v.at[j]], semr,
                             add=True)
        for j in range(CA):
            pltpu.make_async_copy(gf_v.at[j], accf_sh.at[dst_v.at[j]],
                                  semf).wait()
            pltpu.make_async_copy(gr_v.at[j], accr_sh.at[src_v.at[j]],
                                  semr).wait()
        plsc.subcore_barrier()

        rf, rr = pair, 2 + pair
        if last:
            pltpu.sync_copy(accf_sh.at[sl_t], tmp_v)
            pltpu.sync_copy(tmp_v, xout_hbm.at[rf, cid, sl_t])
            pltpu.sync_copy(accr_sh.at[sl_t], tmp_v)
            pltpu.sync_copy(tmp_v, xout_hbm.at[rr, cid, sl_t])
        else:
            pltpu.sync_copy(accf_sh, xf_v)
            pltpu.sync_copy(accr_sh, xr_v)
            pltpu.sync_copy(xf_v.at[sl_t], xout_hbm.at[rf, cid, sl_t])
            pltpu.sync_copy(xr_v.at[sl_t], xout_hbm.at[rr, cid, sl_t])
            plsc.subcore_barrier()

    do_pair(0, False)
    do_pair(1, True)


def _dde(topic_cols, srcA, dstA):
    return pl.kernel(
        _dde_body,
        out_type=jax.ShapeDtypeStruct((4, T, NPAD), jnp.float32),
        mesh=plsc.VectorSubcoreMesh(core_axis_name="c", subcore_axis_name="s",
                                    num_cores=NC, num_subcores=NS),
        compiler_params=pltpu.CompilerParams(needs_layout_passes=False),
        scratch_types=[
            pltpu.VMEM((CA, 128), jnp.int32),
            pltpu.VMEM((CA, 128), jnp.int32),
            pltpu.VMEM((NPAD,), jnp.float32),
            pltpu.VMEM((NPAD,), jnp.float32),
            pltpu.VMEM((CA, 128), jnp.float32),
            pltpu.VMEM((CA, 128), jnp.float32),
            pltpu.VMEM((NPT,), jnp.float32),
            pltpu.VMEM((NPT,), jnp.float32),
            pltpu.VMEM_SHARED((NPAD,), jnp.float32),
            pltpu.VMEM_SHARED((NPAD,), jnp.float32),
            pltpu.SemaphoreType.DMA,
            pltpu.SemaphoreType.DMA,
        ],
    )(topic_cols, srcA, dstA)


def _sem_body(nodes_ref, We_ref, be_ref, sem_ref):
    f32 = jnp.float32
    sem_ref[...] = jnp.dot(nodes_ref[...], We_ref[...],
                           preferred_element_type=f32) + be_ref[...]


def _sem(nodes_pad, We, be2):
    RB = 1024
    return pl.pallas_call(
        _sem_body,
        grid=(NPAD // RB,),
        in_specs=[
            pl.BlockSpec((RB, D), lambda i: (i, 0)),
            pl.BlockSpec((D, D), lambda i: (0, 0)),
            pl.BlockSpec((1, D), lambda i: (0, 0)),
        ],
        out_specs=pl.BlockSpec((RB, D), lambda i: (i, 0)),
        out_shape=jax.ShapeDtypeStruct((NPAD, D), jnp.float32),
    )(nodes_pad, We, be2)


def _dense_body(sem_ref, topic_ref, xout_ref, batch_ref, qemb_ref,
                Wq_ref, bq_ref, Wr_ref, br_ref,
                Wf_ref, bf_ref, G_ref, uc_ref):
    f32 = jnp.float32
    semantic = sem_ref[...]
    film = jnp.dot(topic_ref[...], Wf_ref[0, :T, :],
                   preferred_element_type=f32) + bf_ref[...]
    for r in range(4):
        film = film + lax.dot_general(
            xout_ref[r], Wf_ref[0, T + 2 * r:T + 2 * r + 2, :],
            (((0,), (0,)), ((), ())), preferred_element_type=f32)
    gamma = film[:, :D]
    beta = film[:, D:]
    enh = gamma * semantic + beta
    q = jnp.dot(qemb_ref[...], Wq_ref[...], preferred_element_type=f32) \
        + bq_ref[...]
    bvec = batch_ref[...]
    qs = jnp.zeros_like(enh)
    for b in range(B):
        qs = jnp.where(bvec == b, q[b][None, :], qs)
    h = D // 2
    qsr, qsi = qs[:, :h], qs[:, h:]
    er, eim = enh[:, :h], enh[:, h:]
    g = jnp.concatenate([qsr * er + qsi * eim, qsi * er - qsr * eim], axis=1)
    Gfull = lax.dot_general(g, Wr_ref[...], (((1,), (1,)), ((), ())),
                            preferred_element_type=f32)
    h2 = D // 2
    ai = lax.bitcast_convert_type(Gfull[:, :h2].astype(jnp.bfloat16),
                                  jnp.uint16).astype(jnp.int32)
    bi = lax.bitcast_convert_type(Gfull[:, h2:].astype(jnp.bfloat16),
                                  jnp.uint16).astype(jnp.int32)
    G_ref[...] = lax.bitcast_convert_type(
        jnp.bitwise_or(ai, jnp.left_shift(bi, 16)), f32)
    u = lax.dot_general(enh, q, (((1,), (1,)), ((), ())),
                        preferred_element_type=f32)
    c = lax.dot_general(g, br_ref[...], (((1,), (1,)), ((), ())),
                        preferred_element_type=f32)
    uc_ref[...] = jnp.concatenate(
        [u, c, jnp.zeros((u.shape[0], 8 - B - 1), f32)], axis=1)


def _dense(sem_pad, topic_pad, xout, batch_pad2, qemb,
           Wq, bq2, Wr, br2, Wf, bf2):
    RB = 1024
    nb = NPAD // RB
    full = lambda shape: pl.BlockSpec(shape, lambda i: (0, 0))
    blk = lambda shape: pl.BlockSpec(shape, lambda i: (i, 0))
    return pl.pallas_call(
        _dense_body,
        grid=(nb,),
        in_specs=[
            blk((RB, D)), blk((RB, T)),
            pl.BlockSpec((4, T, RB), lambda i: (0, 0, i)),
            blk((RB, 1)), full((B, D)),
            full((D, D)), full((1, D)),
            full((D, D)), full((1, D)),
            pl.BlockSpec((1, SD, 2 * D), lambda i: (0, 0, 0)),
            full((1, 2 * D)),
        ],
        out_specs=[blk((RB, D // 2)), blk((RB, 8))],
        out_shape=[
            jax.ShapeDtypeStruct((NPAD, D // 2), jnp.float32),
            jax.ShapeDtypeStruct((NPAD, 8), jnp.float32),
        ],
    )(sem_pad, topic_pad, xout, batch_pad2, qemb,
      Wq, bq2, Wr, br2, Wf.reshape(1, SD, 2 * D), bf2)


def _score_body(G_hbm, srcC_hbm, dstC_hbm, batch_hbm, cpad_hbm, utab_hbm,
                ghat_hbm, adj_hbm,
                src_v, dst_v, batch_v, c_v, utab_v, gbuf, adjbuf,
                sem_in, sem_out):
    cid = lax.axis_index("c")
    sid = lax.axis_index("s")
    wid = sid * NC + cid

    pltpu.sync_copy(srcC_hbm.at[wid], src_v)
    pltpu.sync_copy(dstC_hbm.at[wid], dst_v)
    pltpu.sync_copy(batch_hbm, batch_v)
    pltpu.sync_copy(cpad_hbm, c_v)
    pltpu.sync_copy(utab_hbm, utab_v)

    def ebase(j):
        return wid * EPW + j * KC

    def fire(j):
        b = j & 1
        pltpu.async_copy(G_hbm.at[src_v.at[j]], gbuf.at[b], sem_in.at[b])

    def wait_in(j):
        b = j & 1
        pltpu.make_async_copy(G_hbm.at[src_v.at[j]], gbuf.at[b],
                              sem_in.at[b]).wait()

    def wait_out(j):
        b = j & 1

        @pl.when(ebase(j) < E)
        def _():
            pltpu.make_async_copy(gbuf.at[b],
                                  ghat_hbm.at[pl.ds(ebase(j), KC)],
                                  sem_out.at[b]).wait()
            pltpu.make_async_copy(adjbuf.at[b],
                                  adj_hbm.at[pl.ds(ebase(j), KC)],
                                  sem_out.at[b]).wait()

    def compute(j):
        b = j & 1
        for g in range(KC // L):
            sl = pl.ds(g * L, L)
            srcv = src_v[j, sl]
            dstv = dst_v[j, sl]
            qid = plsc.load_gather(batch_v, [srcv])
            pw = plsc.load_gather(utab_v, [dstv * 2 + (qid >> 1)])
            ue, uo = plsc.unpack(plsc.bitcast(pw, jnp.bfloat16),
                                 format=plsc.PackFormat.INTERLEAVED)
            uval = jnp.where((qid & 1) == 1, uo, ue)
            adjbuf[b, sl] = plsc.load_gather(c_v, [srcv]) - uval

        @pl.when(ebase(j) < E)
        def _():
            pltpu.async_copy(gbuf.at[b], ghat_hbm.at[pl.ds(ebase(j), KC)],
                             sem_out.at[b])
            pltpu.async_copy(adjbuf.at[b], adj_hbm.at[pl.ds(ebase(j), KC)],
                             sem_out.at[b])

    fire(0)

    def loop_body(j, carry):
        @pl.when(j >= 1)
        def _():
            wait_out(j - 1)

        @pl.when(j + 1 < NCH)
        def _():
            fire(j + 1)
        wait_in(j)
        compute(j)
        return carry
    lax.fori_loop(0, NCH, loop_body, 0)
    wait_out(NCH - 1)


def _score(G_pad, srcC, dstC, batch_pad, cpad, utab):
    return pl.kernel(
        _score_body,
        out_type=[
            jax.ShapeDtypeStruct((E, D // 2), jnp.float32),
            jax.ShapeDtypeStruct((E,), jnp.float32),
        ],
        mesh=plsc.VectorSubcoreMesh(core_axis_name="c", subcore_axis_name="s",
                                    num_cores=NC, num_subcores=NS),
        compiler_params=pltpu.CompilerParams(needs_layout_passes=False),
        scratch_types=[
            pltpu.VMEM((NCH, KC), jnp.int32),
            pltpu.VMEM((NCH, KC), jnp.int32),
            pltpu.VMEM((NTAB,), jnp.int32),
            pltpu.VMEM((NTAB,), jnp.float32),
            pltpu.VMEM((NTAB * 2,), jnp.float32),
            pltpu.VMEM((2, KC, D // 2), jnp.float32),
            pltpu.VMEM((2, KC), jnp.float32),
            pltpu.SemaphoreType.DMA((2,)),
            pltpu.SemaphoreType.DMA((2,)),
        ],
    )(G_pad, srcC, dstC, batch_pad, cpad, utab)


def _dot_body(ee_ref, gh_ref, adj_ref, out_ref):
    f32 = jnp.float32
    gi = lax.bitcast_convert_type(gh_ref[...], jnp.int32)
    lo = lax.bitcast_convert_type(
        jnp.bitwise_and(gi, 0xFFFF).astype(jnp.uint16),
        jnp.bfloat16).astype(f32)
    hi = lax.bitcast_convert_type(
        lax.shift_right_logical(gi, 16).astype(jnp.uint16),
        jnp.bfloat16).astype(f32)
    ee = ee_ref[...]
    h2 = D // 2
    prod = ee[:, :h2] * lo + ee[:, h2:] * hi
    out_ref[...] = jnp.sum(prod, axis=1, keepdims=True) + adj_ref[...]


def _dot(ee, ghat, adj2):
    RE = 2000
    return pl.pallas_call(
        _dot_body,
        grid=(E // RE,),
        in_specs=[
            pl.BlockSpec((RE, D), lambda i: (i, 0)),
            pl.BlockSpec((RE, D // 2), lambda i: (i, 0)),
            pl.BlockSpec((RE, 1), lambda i: (i, 0)),
        ],
        out_specs=pl.BlockSpec((RE, 1), lambda i: (i, 0)),
        out_shape=jax.ShapeDtypeStruct((E, 1), jnp.float32),
    )(ee, ghat, adj2)


def kernel(node_embeddings, topic_one_hot, edge_index, reverse_edge_index,
           edge_embeddings, question_emb, batch,
           W_entity, b_entity, W_query, b_query, W_relation, b_relation,
           W_film, b_film):
    i32 = jnp.int32
    f32 = jnp.float32
    src = edge_index[0].astype(i32)
    dst = edge_index[1].astype(i32)

    padn = NPAD - N
    topic_pad = jnp.pad(topic_one_hot.astype(f32), ((0, padn), (0, 0)))
    topic_cols = topic_pad.T
    fillA = jnp.full((EPAD_A - E,), N, i32)
    srcA = jnp.concatenate([src, fillA]).reshape(NS, CA, 128)
    dstA = jnp.concatenate([dst, fillA]).reshape(NS, CA, 128)
    xout = _dde(topic_cols, srcA, dstA)

    nodes_pad = jnp.pad(node_embeddings, ((0, padn), (0, 0)))
    sem_pad = _sem(nodes_pad, W_entity, b_entity.reshape(1, D))
    batch_pad = jnp.pad(batch.astype(i32), (0, padn))
    G_pad, uc = _dense(
        sem_pad, topic_pad, xout, batch_pad.reshape(NPAD, 1), question_emb,
        W_query, b_query.reshape(1, D),
        W_relation, b_relation.reshape(1, D), W_film,
        b_film.reshape(1, 2 * D))

    utab = lax.bitcast_convert_type(
        uc[:NTAB, :B].astype(jnp.bfloat16).reshape(NTAB, 2, 2),
        jnp.float32).reshape(NTAB * 2)
    ctab = uc[:NTAB, B]
    fillC = jnp.full((EPAD_C - E,), N, i32)
    srcC = jnp.concatenate([src, fillC]).reshape(NW, NCH, KC)
    dstC = jnp.concatenate([dst, fillC]).reshape(NW, NCH, KC)
    ghat, adj = _score(G_pad, srcC, dstC, batch_pad[:NTAB], ctab, utab)
    return _dot(edge_embeddings, ghat, adj.reshape(E, 1)).reshape(E)


# speedup vs baseline: 1.6030x; 1.6030x over previous
import functools

import numpy as np

import jax
import jax.numpy as jnp
from jax import lax
from jax.experimental import pallas as pl
from jax.experimental.pallas import tpu as pltpu
from jax.experimental.pallas import tpu_sc as plsc

N = 10000
E = 160000
D = 256
T = 2
B = 4
SD = 10

NC = 2
NS = 16
L = 16

NPAD = 10240

NPT = NPAD // NS

CA = 79
EPT_A = CA * 128
EPAD_A = NS * EPT_A

KC = 64
NW = NC * NS
NCH = 79
EPW = NCH * KC
EPAD_C = NW * EPW
NTAB = 10048


def _dde_body(topic_hbm, src_hbm, dst_hbm, xout_hbm,
              src_v, dst_v, xf_v, xr_v, gf_v, gr_v, zero_v, tmp_v,
              accf_sh, accr_sh, semf, semr):
    cid = lax.axis_index("c")
    sid = lax.axis_index("s")
    pltpu.sync_copy(src_hbm.at[sid], src_v)
    pltpu.sync_copy(dst_hbm.at[sid], dst_v)
    for i in range(NPT // L):
        zero_v[pl.ds(i * L, L)] = jnp.zeros((L,), jnp.float32)
    pltpu.sync_copy(topic_hbm.at[cid], xf_v)
    pltpu.sync_copy(topic_hbm.at[cid], xr_v)

    def do_pair(pair, last):
        sl_t = pl.ds(sid * NPT, NPT)
        pltpu.sync_copy(zero_v, accf_sh.at[sl_t])
        pltpu.sync_copy(zero_v, accr_sh.at[sl_t])
        plsc.subcore_barrier()

        def gbody(i, carry):
            for k in range(8):
                sl = pl.ds(k * L, L)
                gf_v[i, sl] = plsc.load_gather(xf_v, [src_v[i, sl]])
                gr_v[i, sl] = plsc.load_gather(xr_v, [dst_v[i, sl]])
            return carry
        lax.fori_loop(0, CA, gbody, 0)

        for j in range(CA):
            pltpu.async_copy(gf_v.at[j], accf_sh.at[dst_v.at[j]], semf,
                             add=True)
            pltpu.async_copy(gr_v.at[j], accr_sh.at[src_v.at[j]], semr,
                             add=True)
        for j in range(CA):
            pltpu.make_async_copy(gf_v.at[j], accf_sh.at[dst_v.at[j]],
                                  semf).wait()
            pltpu.make_async_copy(gr_v.at[j], accr_sh.at[src_v.at[j]],
                                  semr).wait()
        plsc.subcore_barrier()

        rf, rr = pair, 2 + pair
        if last:
            pltpu.sync_copy(accf_sh.at[sl_t], tmp_v)
            pltpu.sync_copy(tmp_v, xout_hbm.at[rf, cid, sl_t])
            pltpu.sync_copy(accr_sh.at[sl_t], tmp_v)
            pltpu.sync_copy(tmp_v, xout_hbm.at[rr, cid, sl_t])
        else:
            pltpu.sync_copy(accf_sh, xf_v)
            pltpu.sync_copy(accr_sh, xr_v)
            pltpu.sync_copy(xf_v.at[sl_t], xout_hbm.at[rf, cid, sl_t])
            pltpu.sync_copy(xr_v.at[sl_t], xout_hbm.at[rr, cid, sl_t])
            plsc.subcore_barrier()

    do_pair(0, False)
    do_pair(1, True)


def _dde(topic_cols, srcA, dstA):
    return pl.kernel(
        _dde_body,
        out_type=jax.ShapeDtypeStruct((4, T, NPAD), jnp.float32),
        mesh=plsc.VectorSubcoreMesh(core_axis_name="c", subcore_axis_name="s",
                                    num_cores=NC, num_subcores=NS),
        compiler_params=pltpu.CompilerParams(needs_layout_passes=False),
        scratch_types=[
            pltpu.VMEM((CA, 128), jnp.int32),
            pltpu.VMEM((CA, 128), jnp.int32),
            pltpu.VMEM((NPAD,), jnp.float32),
            pltpu.VMEM((NPAD,), jnp.float32),
            pltpu.VMEM((CA, 128), jnp.float32),
            pltpu.VMEM((CA, 128), jnp.float32),
            pltpu.VMEM((NPT,), jnp.float32),
            pltpu.VMEM((NPT,), jnp.float32),
            pltpu.VMEM_SHARED((NPAD,), jnp.float32),
            pltpu.VMEM_SHARED((NPAD,), jnp.float32),
            pltpu.SemaphoreType.DMA,
            pltpu.SemaphoreType.DMA,
        ],
    )(topic_cols, srcA, dstA)


def _sem_body(nodes_ref, We_ref, be_ref, sem_ref):
    f32 = jnp.float32
    sem_ref[...] = jnp.dot(nodes_ref[...], We_ref[...],
                           preferred_element_type=f32) + be_ref[...]


def _sem(nodes_pad, We, be2):
    RB = 1024
    return pl.pallas_call(
        _sem_body,
        grid=(NPAD // RB,),
        in_specs=[
            pl.BlockSpec((RB, D), lambda i: (i, 0)),
            pl.BlockSpec((D, D), lambda i: (0, 0)),
            pl.BlockSpec((1, D), lambda i: (0, 0)),
        ],
        out_specs=pl.BlockSpec((RB, D), lambda i: (i, 0)),
        out_shape=jax.ShapeDtypeStruct((NPAD, D), jnp.float32),
    )(nodes_pad, We, be2)


def _dense_body(sem_ref, topic_ref, xout_ref, batch_ref, qemb_ref,
                Wq_ref, bq_ref, Wr_ref, br_ref,
                Wf_ref, bf_ref, G_ref, uc_ref):
    f32 = jnp.float32
    semantic = sem_ref[...]
    film = jnp.dot(topic_ref[...], Wf_ref[0, :T, :],
                   preferred_element_type=f32) + bf_ref[...]
    for r in range(4):
        film = film + lax.dot_general(
            xout_ref[r], Wf_ref[0, T + 2 * r:T + 2 * r + 2, :],
            (((0,), (0,)), ((), ())), preferred_element_type=f32)
    gamma = film[:, :D]
    beta = film[:, D:]
    enh = gamma * semantic + beta
    q = jnp.dot(qemb_ref[...], Wq_ref[...], preferred_element_type=f32) \
        + bq_ref[...]
    bvec = batch_ref[...]
    qs = jnp.zeros_like(enh)
    for b in range(B):
        qs = jnp.where(bvec == b, q[b][None, :], qs)
    h = D // 2
    qsr, qsi = qs[:, :h], qs[:, h:]
    er, eim = enh[:, :h], enh[:, h:]
    g = jnp.concatenate([qsr * er + qsi * eim, qsi * er - qsr * eim], axis=1)
    Gfull = lax.dot_general(g, Wr_ref[...], (((1,), (1,)), ((), ())),
                            preferred_element_type=f32)
    h2 = D // 2
    ai = lax.bitcast_convert_type(Gfull[:, :h2].astype(jnp.bfloat16),
                                  jnp.uint16).astype(jnp.int32)
    bi = lax.bitcast_convert_type(Gfull[:, h2:].astype(jnp.bfloat16),
                                  jnp.uint16).astype(jnp.int32)
    G_ref[...] = lax.bitcast_convert_type(
        jnp.bitwise_or(ai, jnp.left_shift(bi, 16)), f32)
    u = lax.dot_general(enh, q, (((1,), (1,)), ((), ())),
                        preferred_element_type=f32)
    c = lax.dot_general(g, br_ref[...], (((1,), (1,)), ((), ())),
                        preferred_element_type=f32)
    uc_ref[...] = jnp.concatenate(
        [u, c, jnp.zeros((u.shape[0], 8 - B - 1), f32)], axis=1)


def _dense(sem_pad, topic_pad, xout, batch_pad2, qemb,
           Wq, bq2, Wr, br2, Wf, bf2):
    RB = 1024
    nb = NPAD // RB
    full = lambda shape: pl.BlockSpec(shape, lambda i: (0, 0))
    blk = lambda shape: pl.BlockSpec(shape, lambda i: (i, 0))
    return pl.pallas_call(
        _dense_body,
        grid=(nb,),
        in_specs=[
            blk((RB, D)), blk((RB, T)),
            pl.BlockSpec((4, T, RB), lambda i: (0, 0, i)),
            blk((RB, 1)), full((B, D)),
            full((D, D)), full((1, D)),
            full((D, D)), full((1, D)),
            pl.BlockSpec((1, SD, 2 * D), lambda i: (0, 0, 0)),
            full((1, 2 * D)),
        ],
        out_specs=[blk((RB, D // 2)), blk((RB, 8))],
        out_shape=[
            jax.ShapeDtypeStruct((NPAD, D // 2), jnp.float32),
            jax.ShapeDtypeStruct((NPAD, 8), jnp.float32),
        ],
    )(sem_pad, topic_pad, xout, batch_pad2, qemb,
      Wq, bq2, Wr, br2, Wf.reshape(1, SD, 2 * D), bf2)


def _score_body(G_hbm, ee_hbm, utab_hbm, srcC_hbm, dstC_hbm,
                batch_hbm, cpad_hbm, out_hbm,
                src_v, dst_v, batch_v, c_v, utab_v,
                gbuf, eebuf, sco_v, tile_v, sem_in, sem_out):
    cid = lax.axis_index("c")
    sid = lax.axis_index("s")
    wid = sid * NC + cid

    pltpu.sync_copy(srcC_hbm.at[wid], src_v)
    pltpu.sync_copy(dstC_hbm.at[wid], dst_v)
    pltpu.sync_copy(batch_hbm, batch_v)
    pltpu.sync_copy(cpad_hbm, c_v)
    pltpu.sync_copy(utab_hbm, utab_v)

    def ee_base(j):
        return jnp.minimum(wid * EPW + j * KC, E - KC)

    def fire(j):
        b = j & 1
        pltpu.async_copy(G_hbm.at[src_v.at[j]], gbuf.at[b], sem_in.at[b])
        pltpu.async_copy(ee_hbm.at[pl.ds(ee_base(j), KC)], eebuf.at[b],
                         sem_in.at[b])

    def wait_in(j):
        b = j & 1
        pltpu.make_async_copy(G_hbm.at[src_v.at[j]], gbuf.at[b],
                              sem_in.at[b]).wait()
        pltpu.make_async_copy(ee_hbm.at[pl.ds(ee_base(j), KC)],
                              eebuf.at[b], sem_in.at[b]).wait()

    def out_wait(j):
        b = j & 1
        pltpu.make_async_copy(sco_v.at[b],
                              out_hbm.at[pl.ds((wid * NCH + j) * KC, KC)],
                              sem_out.at[b]).wait()

    def compute(j):
        b = j & 1

        @pl.when(j >= 2)
        def _():
            out_wait(j - 2)

        @plsc.parallel_loop(0, KC, 1, unroll=4)
        def _dots(e):
            parts = []
            for k in range(8):
                g0, g1 = plsc.unpack(
                    plsc.bitcast(gbuf[b, e, pl.ds(k * L, L)],
                                 jnp.bfloat16),
                    format=plsc.PackFormat.INTERLEAVED)
                t0 = g0 * eebuf[b, e, pl.ds(k * L, L)]
                t1 = g1 * eebuf[b, e, pl.ds(D // 2 + k * L, L)]
                parts.append(t0 + t1)
            q0 = (parts[0] + parts[1]) + (parts[2] + parts[3])
            q1 = (parts[4] + parts[5]) + (parts[6] + parts[7])
            tile_v[e, :] = q0 + q1

        rows = lax.iota(jnp.int32, L)
        for g in range(KC // L):
            ridx = rows + (g * L)
            cols = [plsc.load_gather(tile_v, [ridx, jnp.full((L,), k,
                                                            jnp.int32)])
                    for k in range(L)]
            while len(cols) > 1:
                cols = [cols[i] + cols[i + 1] for i in range(0, len(cols), 2)]
            svec = cols[0]
            srcv = src_v[j, pl.ds(g * L, L)]
            dstv = dst_v[j, pl.ds(g * L, L)]
            qid = plsc.load_gather(batch_v, [srcv])
            pw = plsc.load_gather(utab_v, [dstv * 2 + (qid >> 1)])
            ue, uo = plsc.unpack(plsc.bitcast(pw, jnp.bfloat16),
                                 format=plsc.PackFormat.INTERLEAVED)
            uval = jnp.where((qid & 1) == 1, uo, ue)
            svec = svec + plsc.load_gather(c_v, [srcv]) - uval
            sco_v[b, pl.ds(g * L, L)] = svec
        pltpu.async_copy(sco_v.at[b],
                         out_hbm.at[pl.ds((wid * NCH + j) * KC, KC)],
                         sem_out.at[b])

    fire(0)

    def loop_body(j, carry):
        @pl.when(j + 1 < NCH)
        def _():
            fire(j + 1)
        wait_in(j)
        compute(j)
        return carry
    lax.fori_loop(0, NCH, loop_body, 0)
    out_wait(NCH - 2)
    out_wait(NCH - 1)


def _score(G_pad, ee, uflat, srcC, dstC, batch_pad, cpad):
    return pl.kernel(
        _score_body,
        out_type=jax.ShapeDtypeStruct((EPAD_C,), jnp.float32),
        mesh=plsc.VectorSubcoreMesh(core_axis_name="c", subcore_axis_name="s",
                                    num_cores=NC, num_subcores=NS),
        compiler_params=pltpu.CompilerParams(needs_layout_passes=False),
        scratch_types=[
            pltpu.VMEM((NCH, KC), jnp.int32),
            pltpu.VMEM((NCH, KC), jnp.int32),
            pltpu.VMEM((NTAB,), jnp.int32),
            pltpu.VMEM((NTAB,), jnp.float32),
            pltpu.VMEM((NTAB * 2,), jnp.float32),
            pltpu.VMEM((2, KC, D // 2), jnp.float32),
            pltpu.VMEM((2, KC, D), jnp.float32),
            pltpu.VMEM((2, KC), jnp.float32),
            pltpu.VMEM((KC, L), jnp.float32),
            pltpu.SemaphoreType.DMA((2,)),
            pltpu.SemaphoreType.DMA((2,)),
        ],
    )(G_pad, ee, uflat, srcC, dstC, batch_pad, cpad)


def kernel(node_embeddings, topic_one_hot, edge_index, reverse_edge_index,
           edge_embeddings, question_emb, batch,
           W_entity, b_entity, W_query, b_query, W_relation, b_relation,
           W_film, b_film):
    i32 = jnp.int32
    f32 = jnp.float32
    src = edge_index[0].astype(i32)
    dst = edge_index[1].astype(i32)

    padn = NPAD - N
    topic_pad = jnp.pad(topic_one_hot.astype(f32), ((0, padn), (0, 0)))
    topic_cols = topic_pad.T
    fillA = jnp.full((EPAD_A - E,), N, i32)
    srcA = jnp.concatenate([src, fillA]).reshape(NS, CA, 128)
    dstA = jnp.concatenate([dst, fillA]).reshape(NS, CA, 128)
    xout = _dde(topic_cols, srcA, dstA)

    nodes_pad = jnp.pad(node_embeddings, ((0, padn), (0, 0)))
    sem_pad = _sem(nodes_pad, W_entity, b_entity.reshape(1, D))
    batch_pad = jnp.pad(batch.astype(i32), (0, padn))
    G_pad, uc = _dense(
        sem_pad, topic_pad, xout, batch_pad.reshape(NPAD, 1), question_emb,
        W_query, b_query.reshape(1, D),
        W_relation, b_relation.reshape(1, D), W_film,
        b_film.reshape(1, 2 * D))

    utab = lax.bitcast_convert_type(
        uc[:NTAB, :B].astype(jnp.bfloat16).reshape(NTAB, 2, 2),
        jnp.float32).reshape(NTAB * 2)
    ctab = uc[:NTAB, B]
    fillC = jnp.full((EPAD_C - E,), N, i32)
    srcC = jnp.concatenate([src, fillC]).reshape(NW, NCH, KC)
    dstC = jnp.concatenate([dst, fillC]).reshape(NW, NCH, KC)
    scores = _score(G_pad, edge_embeddings, utab,
                    srcC, dstC, batch_pad[:NTAB], ctab)
    return scores[:E]
